# Initial kernel scaffold; baseline (speedup 1.0000x reference)
import jax
import jax.numpy as jnp
from jax.experimental import pallas as pl


def kernel(x, edge_index, W1, b1, gamma, beta, alpha, W2, b2, Wg, bg, Ww, bw):
    raise NotImplementedError("write your pallas kernel here")


# speedup vs baseline: 160.4246x; 160.4246x over previous
import jax
import jax.numpy as jnp
from jax import lax
from jax.experimental import pallas as pl
from jax.experimental.pallas import tpu as pltpu
from jax.experimental.pallas import tpu_sc as plsc

N = 100000
E = 1600000
H = 32
NS = 16
NP = 100352
SL = NP // NS
EP = E // NS
EC = 10000
ROWS = NP // 128

_f32 = jnp.float32


def _mesh():
    return plsc.VectorSubcoreMesh(
        core_axis_name="c", subcore_axis_name="s", num_cores=1)


def _deg_body(dst_hbm, deg_hbm, didx_v, vals_v, node_a, deg_sh):
    s = lax.axis_index("s")
    nbase = pl.multiple_of(s * SL, SL)
    ebase = pl.multiple_of(s * EP, EP)

    def init_i(i, _):
        vals_v[pl.ds(i * 16, 16)] = jnp.full((16,), 1.0, _f32)
        return 0
    lax.fori_loop(0, EC // 16, init_i, 0)

    def zero_i(i, _):
        node_a[pl.ds(i * 16, 16)] = jnp.zeros((16,), _f32)
        return 0
    lax.fori_loop(0, SL // 16, zero_i, 0)

    pltpu.sync_copy(node_a, deg_sh.at[pl.ds(nbase, SL)])
    plsc.subcore_barrier()

    def deg_step(k, _):
        off = pl.multiple_of(ebase + k * EC, EC)
        pltpu.sync_copy(dst_hbm.at[pl.ds(off, EC)], didx_v)
        pltpu.sync_copy(vals_v, deg_sh.at[didx_v], add=True)
        return 0
    lax.fori_loop(0, EP // EC, deg_step, 0)
    plsc.subcore_barrier()

    pltpu.sync_copy(deg_sh.at[pl.ds(nbase, SL)], node_a)
    pltpu.sync_copy(node_a, deg_hbm.at[pl.ds(nbase, SL)])


def _deg_kernel(dst):
    f = pl.kernel(
        _deg_body,
        out_type=jax.ShapeDtypeStruct((NP,), _f32),
        mesh=_mesh(),
        scratch_types=[
            pltpu.VMEM((EC,), jnp.int32),
            pltpu.VMEM((EC,), _f32),
            pltpu.VMEM((SL,), _f32),
            pltpu.VMEM_SHARED((NP,), _f32),
        ],
    )
    return f(dst)


def _mlp_body(x0_ref, x1_ref, deg_ref, coef_ref, p_ref, dinv_ref):
    x0 = x0_ref[...]
    x1 = x1_ref[...]
    alpha = coef_ref[4 * H]
    cv = coef_ref[4 * H + 1]
    acc = jnp.full_like(x0, cv)
    for j in range(H):
        t = x0 * coef_ref[j] + x1 * coef_ref[H + j] + coef_ref[2 * H + j]
        t = jnp.where(t > 0, t, alpha * t)
        acc = acc + t * coef_ref[3 * H + j]
    d = deg_ref[...] + 1.0
    dinv = lax.rsqrt(d)
    dinv = dinv * (1.5 - 0.5 * d * dinv * dinv)
    dinv_ref[...] = dinv
    p_ref[...] = acc * dinv


def _mlp(x0, x1, deg, coef):
    return pl.pallas_call(
        _mlp_body,
        out_shape=(
            jax.ShapeDtypeStruct((ROWS, 128), _f32),
            jax.ShapeDtypeStruct((ROWS, 128), _f32),
        ),
        in_specs=[
            pl.BlockSpec(memory_space=pltpu.VMEM),
            pl.BlockSpec(memory_space=pltpu.VMEM),
            pl.BlockSpec(memory_space=pltpu.VMEM),
            pl.BlockSpec(memory_space=pltpu.SMEM),
        ],
        out_specs=(
            pl.BlockSpec(memory_space=pltpu.VMEM),
            pl.BlockSpec(memory_space=pltpu.VMEM),
        ),
    )(x0, x1, deg, coef)


def _agg_body(src_hbm, dst_hbm, p_hbm, dinv_hbm, cst_hbm, out_hbm,
              sidx_v, didx_v, vals_v, node_a, node_p, node_d, cst_v,
              p_sh, a_sh):
    s = lax.axis_index("s")
    nbase = pl.multiple_of(s * SL, SL)
    ebase = pl.multiple_of(s * EP, EP)

    def zero_i(i, _):
        node_a[pl.ds(i * 16, 16)] = jnp.zeros((16,), _f32)
        return 0
    lax.fori_loop(0, SL // 16, zero_i, 0)

    pltpu.sync_copy(node_a, a_sh.at[pl.ds(nbase, SL)])
    pltpu.sync_copy(p_hbm.at[pl.ds(nbase, SL)], node_p)
    pltpu.sync_copy(dinv_hbm.at[pl.ds(nbase, SL)], node_d)
    pltpu.sync_copy(cst_hbm, cst_v)
    pltpu.sync_copy(node_p, p_sh.at[pl.ds(nbase, SL)])
    plsc.subcore_barrier()

    def agg_step(k, _):
        off = pl.multiple_of(ebase + k * EC, EC)
        pltpu.sync_copy(src_hbm.at[pl.ds(off, EC)], sidx_v)
        pltpu.sync_copy(dst_hbm.at[pl.ds(off, EC)], didx_v)
        pltpu.sync_copy(p_sh.at[sidx_v], vals_v)
        pltpu.sync_copy(vals_v, a_sh.at[didx_v], add=True)
        return 0
    lax.fori_loop(0, EP // EC, agg_step, 0)
    plsc.subcore_barrier()

    pltpu.sync_copy(a_sh.at[pl.ds(nbase, SL)], node_a)

    def fin_step(i, _):
        sl = pl.ds(pl.multiple_of(i * 16, 16), 16)
        node_a[sl] = node_d[sl] * (node_a[sl] + node_p[sl]) + cst_v[...]
        return 0
    lax.fori_loop(0, SL // 16, fin_step, 0)
    pltpu.sync_copy(node_a, out_hbm.at[pl.ds(nbase, SL)])


def _agg_kernel(src, dst, p, dinv, cst):
    f = pl.kernel(
        _agg_body,
        out_type=jax.ShapeDtypeStruct((NP,), _f32),
        mesh=_mesh(),
        scratch_types=[
            pltpu.VMEM((EC,), jnp.int32),
            pltpu.VMEM((EC,), jnp.int32),
            pltpu.VMEM((EC,), _f32),
            pltpu.VMEM((SL,), _f32),
            pltpu.VMEM((SL,), _f32),
            pltpu.VMEM((SL,), _f32),
            pltpu.VMEM((16,), _f32),
            pltpu.VMEM_SHARED((NP,), _f32),
            pltpu.VMEM_SHARED((NP,), _f32),
        ],
    )
    return f(src, dst, p, dinv, cst)


@jax.jit
def kernel(x, edge_index, W1, b1, gamma, beta, alpha, W2, b2, Wg, bg, Ww, bw):
    wgw = Wg @ Ww
    wv = (W2 @ wgw)[:, 0]
    cv = (b2 @ wgw)[0]
    const = (bg @ Ww)[0] + bw[0]
    a0 = gamma * W1[0]
    a1 = gamma * W1[1]
    cc = gamma * b1 + beta
    coef = jnp.concatenate([
        a0, a1, cc, wv,
        jnp.reshape(alpha, (1,)), jnp.reshape(cv, (1,)),
    ]).astype(_f32)

    deg = _deg_kernel(edge_index[1])

    x0 = jnp.pad(x[:, 0], (0, NP - N)).reshape(ROWS, 128)
    x1 = jnp.pad(x[:, 1], (0, NP - N)).reshape(ROWS, 128)
    p, dinv = _mlp(x0, x1, deg.reshape(ROWS, 128), coef)

    cst = jnp.full((16,), const, _f32)
    scores = _agg_kernel(edge_index[0], edge_index[1],
                         p.reshape(NP), dinv.reshape(NP), cst)
    return scores[:N]


# speedup vs baseline: 281.2007x; 1.7529x over previous
import jax
import jax.numpy as jnp
from jax import lax
from jax.experimental import pallas as pl
from jax.experimental.pallas import tpu as pltpu
from jax.experimental.pallas import tpu_sc as plsc

N = 100000
E = 1600000
H = 32
NC = 2
NS = 16
NW = NC * NS
NP = 100352
SL = NP // NS
EP = E // NW
EC = 25000
ROWS = NP // 128

_f32 = jnp.float32


def _mesh():
    return plsc.VectorSubcoreMesh(
        core_axis_name="c", subcore_axis_name="s", num_cores=NC)


def _deg_body(ei_hbm, deg_hbm, didx_v, vals_v, node_a, deg_sh):
    c = lax.axis_index("c")
    s = lax.axis_index("s")
    nbase = pl.multiple_of(s * SL, SL)
    ebase = pl.multiple_of((c * NS + s) * EP, EP)

    def init_i(i, _):
        vals_v[pl.ds(i * 16, 16)] = jnp.full((16,), 1.0, _f32)
        return 0
    lax.fori_loop(0, EC // 16, init_i, 0)

    def zero_i(i, _):
        node_a[pl.ds(i * 16, 16)] = jnp.zeros((16,), _f32)
        return 0
    lax.fori_loop(0, SL // 16, zero_i, 0)

    pltpu.sync_copy(node_a, deg_sh.at[pl.ds(nbase, SL)])
    plsc.subcore_barrier()

    def deg_step(k, _):
        off = pl.multiple_of(ebase + k * EC, EC)
        pltpu.sync_copy(ei_hbm.at[pl.ds(E + off, EC)], didx_v)
        pltpu.sync_copy(vals_v, deg_sh.at[didx_v], add=True)
        return 0
    lax.fori_loop(0, EP // EC, deg_step, 0)
    plsc.subcore_barrier()

    pltpu.sync_copy(deg_sh.at[pl.ds(nbase, SL)], node_a)
    pltpu.sync_copy(node_a, deg_hbm.at[c, pl.ds(nbase, SL)])


def _deg_kernel(edge_index):
    f = pl.kernel(
        _deg_body,
        out_type=jax.ShapeDtypeStruct((NC, NP), _f32),
        mesh=_mesh(),
        scratch_types=[
            pltpu.VMEM((EC,), jnp.int32),
            pltpu.VMEM((EC,), _f32),
            pltpu.VMEM((SL,), _f32),
            pltpu.VMEM_SHARED((NP,), _f32),
        ],
    )
    return f(edge_index)


def _mlp_body(x0_ref, x1_ref, deg_ref, coef_ref, p_ref, dinv_ref):
    x0 = x0_ref[...]
    x1 = x1_ref[...]
    alpha = coef_ref[4 * H]
    cv = coef_ref[4 * H + 1]
    acc = jnp.full_like(x0, cv)
    for j in range(H):
        t = x0 * coef_ref[j] + x1 * coef_ref[H + j] + coef_ref[2 * H + j]
        t = jnp.where(t > 0, t, alpha * t)
        acc = acc + t * coef_ref[3 * H + j]
    d = deg_ref[0] + deg_ref[1] + 1.0
    dinv = lax.rsqrt(d)
    dinv = dinv * (1.5 - 0.5 * d * dinv * dinv)
    dinv_ref[...] = dinv
    p_ref[...] = acc * dinv


def _mlp(x0, x1, deg, coef):
    return pl.pallas_call(
        _mlp_body,
        out_shape=(
            jax.ShapeDtypeStruct((ROWS, 128), _f32),
            jax.ShapeDtypeStruct((ROWS, 128), _f32),
        ),
        in_specs=[
            pl.BlockSpec(memory_space=pltpu.VMEM),
            pl.BlockSpec(memory_space=pltpu.VMEM),
            pl.BlockSpec(memory_space=pltpu.VMEM),
            pl.BlockSpec(memory_space=pltpu.SMEM),
        ],
        out_specs=(
            pl.BlockSpec(memory_space=pltpu.VMEM),
            pl.BlockSpec(memory_space=pltpu.VMEM),
        ),
    )(x0, x1, deg, coef)


def _agg_body(ei_hbm, p_hbm, a_hbm, sidx_v, didx_v, vals_v, node_a, p_sh, a_sh):
    c = lax.axis_index("c")
    s = lax.axis_index("s")
    nbase = pl.multiple_of(s * SL, SL)
    ebase = pl.multiple_of((c * NS + s) * EP, EP)

    def zero_i(i, _):
        node_a[pl.ds(i * 16, 16)] = jnp.zeros((16,), _f32)
        return 0
    lax.fori_loop(0, SL // 16, zero_i, 0)

    pltpu.sync_copy(node_a, a_sh.at[pl.ds(nbase, SL)])
    pltpu.sync_copy(p_hbm.at[pl.ds(nbase, SL)], node_a)
    pltpu.sync_copy(node_a, p_sh.at[pl.ds(nbase, SL)])
    plsc.subcore_barrier()

    def agg_step(k, _):
        off = pl.multiple_of(ebase + k * EC, EC)
        pltpu.sync_copy(ei_hbm.at[pl.ds(off, EC)], sidx_v)
        pltpu.sync_copy(ei_hbm.at[pl.ds(E + off, EC)], didx_v)
        pltpu.sync_copy(p_sh.at[sidx_v], vals_v)
        pltpu.sync_copy(vals_v, a_sh.at[didx_v], add=True)
        return 0
    lax.fori_loop(0, EP // EC, agg_step, 0)
    plsc.subcore_barrier()

    pltpu.sync_copy(a_sh.at[pl.ds(nbase, SL)], node_a)
    pltpu.sync_copy(node_a, a_hbm.at[c, pl.ds(nbase, SL)])


def _agg_kernel(edge_index, p):
    f = pl.kernel(
        _agg_body,
        out_type=jax.ShapeDtypeStruct((NC, NP), _f32),
        mesh=_mesh(),
        scratch_types=[
            pltpu.VMEM((EC,), jnp.int32),
            pltpu.VMEM((EC,), jnp.int32),
            pltpu.VMEM((EC,), _f32),
            pltpu.VMEM((SL,), _f32),
            pltpu.VMEM_SHARED((NP,), _f32),
            pltpu.VMEM_SHARED((NP,), _f32),
        ],
    )
    return f(edge_index, p)


def _fin_body(a_ref, p_ref, dinv_ref, cst_ref, out_ref):
    out_ref[...] = dinv_ref[...] * (a_ref[0] + a_ref[1] + p_ref[...]) + cst_ref[0]


def _fin(a_part, p, dinv, cst):
    return pl.pallas_call(
        _fin_body,
        out_shape=jax.ShapeDtypeStruct((ROWS, 128), _f32),
        in_specs=[
            pl.BlockSpec(memory_space=pltpu.VMEM),
            pl.BlockSpec(memory_space=pltpu.VMEM),
            pl.BlockSpec(memory_space=pltpu.VMEM),
            pl.BlockSpec(memory_space=pltpu.SMEM),
        ],
        out_specs=pl.BlockSpec(memory_space=pltpu.VMEM),
    )(a_part, p, dinv, cst)


@jax.jit
def kernel(x, edge_index, W1, b1, gamma, beta, alpha, W2, b2, Wg, bg, Ww, bw):
    wgw = Wg @ Ww
    wv = (W2 @ wgw)[:, 0]
    cv = (b2 @ wgw)[0]
    const = (bg @ Ww)[0] + bw[0]
    a0 = gamma * W1[0]
    a1 = gamma * W1[1]
    cc = gamma * b1 + beta
    coef = jnp.concatenate([
        a0, a1, cc, wv,
        jnp.reshape(alpha, (1,)), jnp.reshape(cv, (1,)),
    ]).astype(_f32)

    ei_flat = edge_index.reshape(2 * E)
    deg_part = _deg_kernel(ei_flat)

    x0 = jnp.pad(x[:, 0], (0, NP - N)).reshape(ROWS, 128)
    x1 = jnp.pad(x[:, 1], (0, NP - N)).reshape(ROWS, 128)
    p, dinv = _mlp(x0, x1, deg_part.reshape(NC, ROWS, 128), coef)

    a_part = _agg_kernel(ei_flat, p.reshape(NP))
    scores = _fin(a_part.reshape(NC, ROWS, 128), p, dinv,
                  jnp.reshape(const, (1,)))
    return scores.reshape(NP)[:N]


# speedup vs baseline: 283.4204x; 1.0079x over previous
import jax
import jax.numpy as jnp
from jax import lax
from jax.experimental import pallas as pl
from jax.experimental.pallas import tpu as pltpu
from jax.experimental.pallas import tpu_sc as plsc

N = 100000
E = 1600000
H = 32
NC = 2
NS = 16
NW = NC * NS
NP = 100352
SL = NP // NS
ECE = 12800
NCH = E // ECE
EP = E // NW
EC = 25000
ROWS = NP // 128

_f32 = jnp.float32


def _mesh():
    return plsc.VectorSubcoreMesh(
        core_axis_name="c", subcore_axis_name="s", num_cores=NC)


def _nchunks(w):
    return 3 + jnp.where(w < NCH - 3 * NW, 1, 0)


def _deg_body(ei_hbm, deg_hbm, src_hbm, dst_hbm,
              eiv, sidx_v, didx_v, vals_v, node_a, deg_sh):
    c = lax.axis_index("c")
    s = lax.axis_index("s")
    w = c * NS + s
    nbase = pl.multiple_of(s * SL, SL)

    def init_i(i, _):
        vals_v[pl.ds(i * 16, 16)] = jnp.full((16,), 1.0, _f32)
        return 0
    lax.fori_loop(0, ECE // 16, init_i, 0)

    def zero_i(i, _):
        node_a[pl.ds(i * 16, 16)] = jnp.zeros((16,), _f32)
        return 0
    lax.fori_loop(0, SL // 16, zero_i, 0)

    pltpu.sync_copy(node_a, deg_sh.at[pl.ds(nbase, SL)])
    plsc.subcore_barrier()

    def deg_step(k, _):
        off = pl.multiple_of((w + NW * k) * ECE, 512)
        pltpu.sync_copy(ei_hbm.at[:, pl.ds(off, ECE)], eiv)

        def dei(i, _):
            sl = pl.ds(pl.multiple_of(i * 16, 16), 16)
            sidx_v[sl] = eiv[0, sl]
            didx_v[sl] = eiv[1, sl]
            return 0
        lax.fori_loop(0, ECE // 16, dei, 0)

        pltpu.sync_copy(sidx_v, src_hbm.at[pl.ds(off, ECE)])
        pltpu.sync_copy(didx_v, dst_hbm.at[pl.ds(off, ECE)])
        pltpu.sync_copy(vals_v, deg_sh.at[didx_v], add=True)
        return 0
    lax.fori_loop(0, _nchunks(w), deg_step, 0)
    plsc.subcore_barrier()

    pltpu.sync_copy(deg_sh.at[pl.ds(nbase, SL)], node_a)
    pltpu.sync_copy(node_a, deg_hbm.at[c, pl.ds(nbase, SL)])


def _deg_kernel(edge_index):
    f = pl.kernel(
        _deg_body,
        out_type=(
            jax.ShapeDtypeStruct((NC, NP), _f32),
            jax.ShapeDtypeStruct((E,), jnp.int32),
            jax.ShapeDtypeStruct((E,), jnp.int32),
        ),
        mesh=_mesh(),
        scratch_types=[
            pltpu.VMEM((2, ECE), jnp.int32),
            pltpu.VMEM((ECE,), jnp.int32),
            pltpu.VMEM((ECE,), jnp.int32),
            pltpu.VMEM((ECE,), _f32),
            pltpu.VMEM((SL,), _f32),
            pltpu.VMEM_SHARED((NP,), _f32),
        ],
    )
    return f(edge_index)


def _mlp_body(x0_ref, x1_ref, coef_ref, v_ref):
    x0 = x0_ref[...]
    x1 = x1_ref[...]
    alpha = coef_ref[4 * H]
    cv = coef_ref[4 * H + 1]
    acc = jnp.full_like(x0, cv)
    for j in range(H):
        t = x0 * coef_ref[j] + x1 * coef_ref[H + j] + coef_ref[2 * H + j]
        t = jnp.where(t > 0, t, alpha * t)
        acc = acc + t * coef_ref[3 * H + j]
    v_ref[...] = acc


def _mlp(x0, x1, coef):
    return pl.pallas_call(
        _mlp_body,
        out_shape=jax.ShapeDtypeStruct((ROWS, 128), _f32),
        in_specs=[
            pl.BlockSpec(memory_space=pltpu.VMEM),
            pl.BlockSpec(memory_space=pltpu.VMEM),
            pl.BlockSpec(memory_space=pltpu.SMEM),
        ],
        out_specs=pl.BlockSpec(memory_space=pltpu.VMEM),
    )(x0, x1, coef)


def _dp_body(deg_ref, v_ref, p_ref, dinv_ref):
    d = deg_ref[0] + deg_ref[1] + 1.0
    dinv = lax.rsqrt(d)
    dinv = dinv * (1.5 - 0.5 * d * dinv * dinv)
    dinv_ref[...] = dinv
    p_ref[...] = v_ref[...] * dinv


def _dp(deg, v):
    return pl.pallas_call(
        _dp_body,
        out_shape=(
            jax.ShapeDtypeStruct((ROWS, 128), _f32),
            jax.ShapeDtypeStruct((ROWS, 128), _f32),
        ),
        in_specs=[
            pl.BlockSpec(memory_space=pltpu.VMEM),
            pl.BlockSpec(memory_space=pltpu.VMEM),
        ],
        out_specs=(
            pl.BlockSpec(memory_space=pltpu.VMEM),
            pl.BlockSpec(memory_space=pltpu.VMEM),
        ),
    )(deg, v)


def _agg_body(src_hbm, dst_hbm, p_hbm, a_hbm, sidx_v, didx_v, vals_v, node_a, p_sh, a_sh):
    c = lax.axis_index("c")
    s = lax.axis_index("s")
    w = c * NS + s
    nbase = pl.multiple_of(s * SL, SL)

    def zero_i(i, _):
        node_a[pl.ds(i * 16, 16)] = jnp.zeros((16,), _f32)
        return 0
    lax.fori_loop(0, SL // 16, zero_i, 0)

    pltpu.sync_copy(node_a, a_sh.at[pl.ds(nbase, SL)])
    pltpu.sync_copy(p_hbm.at[pl.ds(nbase, SL)], node_a)
    pltpu.sync_copy(node_a, p_sh.at[pl.ds(nbase, SL)])
    plsc.subcore_barrier()

    def agg_step(k, _):
        off = pl.multiple_of(w * EP + k * EC, EC)
        pltpu.sync_copy(src_hbm.at[pl.ds(off, EC)], sidx_v)
        pltpu.sync_copy(dst_hbm.at[pl.ds(off, EC)], didx_v)
        pltpu.sync_copy(p_sh.at[sidx_v], vals_v)
        pltpu.sync_copy(vals_v, a_sh.at[didx_v], add=True)
        return 0
    lax.fori_loop(0, EP // EC, agg_step, 0)
    plsc.subcore_barrier()

    pltpu.sync_copy(a_sh.at[pl.ds(nbase, SL)], node_a)
    pltpu.sync_copy(node_a, a_hbm.at[c, pl.ds(nbase, SL)])


def _agg_kernel(src, dst, p):
    f = pl.kernel(
        _agg_body,
        out_type=jax.ShapeDtypeStruct((NC, NP), _f32),
        mesh=_mesh(),
        scratch_types=[
            pltpu.VMEM((EC,), jnp.int32),
            pltpu.VMEM((EC,), jnp.int32),
            pltpu.VMEM((EC,), _f32),
            pltpu.VMEM((SL,), _f32),
            pltpu.VMEM_SHARED((NP,), _f32),
            pltpu.VMEM_SHARED((NP,), _f32),
        ],
    )
    return f(src, dst, p)


def _fin_body(a_ref, p_ref, dinv_ref, cst_ref, out_ref):
    out_ref[...] = dinv_ref[...] * (a_ref[0] + a_ref[1] + p_ref[...]) + cst_ref[0]


def _fin(a_part, p, dinv, cst):
    return pl.pallas_call(
        _fin_body,
        out_shape=jax.ShapeDtypeStruct((ROWS, 128), _f32),
        in_specs=[
            pl.BlockSpec(memory_space=pltpu.VMEM),
            pl.BlockSpec(memory_space=pltpu.VMEM),
            pl.BlockSpec(memory_space=pltpu.VMEM),
            pl.BlockSpec(memory_space=pltpu.SMEM),
        ],
        out_specs=pl.BlockSpec(memory_space=pltpu.VMEM),
    )(a_part, p, dinv, cst)


@jax.jit
def kernel(x, edge_index, W1, b1, gamma, beta, alpha, W2, b2, Wg, bg, Ww, bw):
    wgw = Wg @ Ww
    wv = (W2 @ wgw)[:, 0]
    cv = (b2 @ wgw)[0]
    const = (bg @ Ww)[0] + bw[0]
    a0 = gamma * W1[0]
    a1 = gamma * W1[1]
    cc = gamma * b1 + beta
    coef = jnp.concatenate([
        a0, a1, cc, wv,
        jnp.reshape(alpha, (1,)), jnp.reshape(cv, (1,)),
    ]).astype(_f32)

    x0 = jnp.pad(x[:, 0], (0, NP - N)).reshape(ROWS, 128)
    x1 = jnp.pad(x[:, 1], (0, NP - N)).reshape(ROWS, 128)
    v = _mlp(x0, x1, coef)

    deg_part, src_flat, dst_flat = _deg_kernel(edge_index)
    p, dinv = _dp(deg_part.reshape(NC, ROWS, 128), v)

    a_part = _agg_kernel(src_flat, dst_flat, p.reshape(NP))
    scores = _fin(a_part.reshape(NC, ROWS, 128), p, dinv,
                  jnp.reshape(const, (1,)))
    return scores.reshape(NP)[:N]


# speedup vs baseline: 302.9840x; 1.0690x over previous
import jax
import jax.numpy as jnp
from jax import lax
from jax.experimental import pallas as pl
from jax.experimental.pallas import tpu as pltpu
from jax.experimental.pallas import tpu_sc as plsc

N = 100000
E = 1600000
H = 32
NC = 2
NS = 16
NW = NC * NS
NP = 100352
SL = NP // NS
ECE = 12800
NCH = E // ECE
EP = E // NW
EC = 25000
ROWS = NP // 128

_f32 = jnp.float32


def _mesh():
    return plsc.VectorSubcoreMesh(
        core_axis_name="c", subcore_axis_name="s", num_cores=NC)


def _nchunks(w):
    return 3 + jnp.where(w < NCH - 3 * NW, 1, 0)


def _deg_body(ei_hbm, deg_hbm, src_hbm, dst_hbm,
              eiv0, eiv1, sidx_v, didx_v, vals_v, node_a, deg_sh,
              seml0, seml1, semw):
    c = lax.axis_index("c")
    s = lax.axis_index("s")
    w = c * NS + s
    nbase = pl.multiple_of(s * SL, SL)

    def init_i(i, _):
        vals_v[pl.ds(i * 16, 16)] = jnp.full((16,), 1.0, _f32)
        return 0
    lax.fori_loop(0, ECE // 16, init_i, 0)

    def zero_i(i, _):
        node_a[pl.ds(i * 16, 16)] = jnp.zeros((16,), _f32)
        return 0
    lax.fori_loop(0, SL // 16, zero_i, 0)

    pltpu.sync_copy(node_a, deg_sh.at[pl.ds(nbase, SL)])
    plsc.subcore_barrier()

    nc = _nchunks(w)

    def _off(k):
        return pl.multiple_of((w + NW * k) * ECE, 512)

    def _deint(buf):
        def dei(i, _):
            sl = pl.ds(pl.multiple_of(i * 16, 16), 16)
            sidx_v[sl] = buf[0, sl]
            didx_v[sl] = buf[1, sl]
            return 0
        lax.fori_loop(0, ECE // 16, dei, 0)

    def _tail(k):
        wa = pltpu.async_copy(sidx_v, src_hbm.at[pl.ds(_off(k), ECE)], semw)
        wb = pltpu.async_copy(didx_v, dst_hbm.at[pl.ds(_off(k), ECE)], semw)
        pltpu.sync_copy(vals_v, deg_sh.at[didx_v], add=True)
        wa.wait()
        wb.wait()

    bufs = (eiv0, eiv1)
    sems = (seml0, seml1)
    lds = [pltpu.async_copy(ei_hbm.at[:, pl.ds(_off(k), ECE)], bufs[k], sems[k])
           for k in range(2)]
    lds[0].wait()
    _deint(eiv0)
    ld2 = pltpu.async_copy(ei_hbm.at[:, pl.ds(_off(2), ECE)], eiv0, seml0)
    _tail(0)
    lds[1].wait()
    _deint(eiv1)

    @pl.when(nc == 4)
    def _():
        pltpu.async_copy(ei_hbm.at[:, pl.ds(_off(3), ECE)], eiv1, seml1)
    _tail(1)
    ld2.wait()
    _deint(eiv0)
    _tail(2)

    @pl.when(nc == 4)
    def _():
        pltpu.make_async_copy(
            ei_hbm.at[:, pl.ds(_off(3), ECE)], eiv1, seml1).wait()
        _deint(eiv1)
        _tail(3)
    plsc.subcore_barrier()

    pltpu.sync_copy(deg_sh.at[pl.ds(nbase, SL)], node_a)
    pltpu.sync_copy(node_a, deg_hbm.at[c, pl.ds(nbase, SL)])


def _deg_kernel(edge_index):
    f = pl.kernel(
        _deg_body,
        out_type=(
            jax.ShapeDtypeStruct((NC, NP), _f32),
            jax.ShapeDtypeStruct((E,), jnp.int32),
            jax.ShapeDtypeStruct((E,), jnp.int32),
        ),
        mesh=_mesh(),
        scratch_types=[
            pltpu.VMEM((2, ECE), jnp.int32),
            pltpu.VMEM((2, ECE), jnp.int32),
            pltpu.VMEM((ECE,), jnp.int32),
            pltpu.VMEM((ECE,), jnp.int32),
            pltpu.VMEM((ECE,), _f32),
            pltpu.VMEM((SL,), _f32),
            pltpu.VMEM_SHARED((NP,), _f32),
            pltpu.SemaphoreType.DMA,
            pltpu.SemaphoreType.DMA,
            pltpu.SemaphoreType.DMA,
        ],
    )
    return f(edge_index)


def _mlp_body(x0_ref, x1_ref, coef_ref, v_ref):
    x0 = x0_ref[...]
    x1 = x1_ref[...]
    alpha = coef_ref[4 * H]
    cv = coef_ref[4 * H + 1]
    acc = jnp.full_like(x0, cv)
    for j in range(H):
        t = x0 * coef_ref[j] + x1 * coef_ref[H + j] + coef_ref[2 * H + j]
        t = jnp.where(t > 0, t, alpha * t)
        acc = acc + t * coef_ref[3 * H + j]
    v_ref[...] = acc


def _mlp(x0, x1, coef):
    return pl.pallas_call(
        _mlp_body,
        out_shape=jax.ShapeDtypeStruct((ROWS, 128), _f32),
        in_specs=[
            pl.BlockSpec(memory_space=pltpu.VMEM),
            pl.BlockSpec(memory_space=pltpu.VMEM),
            pl.BlockSpec(memory_space=pltpu.SMEM),
        ],
        out_specs=pl.BlockSpec(memory_space=pltpu.VMEM),
    )(x0, x1, coef)


def _dp_body(deg_ref, v_ref, p_ref, dinv_ref):
    d = deg_ref[0] + deg_ref[1] + 1.0
    dinv = lax.rsqrt(d)
    dinv = dinv * (1.5 - 0.5 * d * dinv * dinv)
    dinv_ref[...] = dinv
    p_ref[...] = v_ref[...] * dinv


def _dp(deg, v):
    return pl.pallas_call(
        _dp_body,
        out_shape=(
            jax.ShapeDtypeStruct((ROWS, 128), _f32),
            jax.ShapeDtypeStruct((ROWS, 128), _f32),
        ),
        in_specs=[
            pl.BlockSpec(memory_space=pltpu.VMEM),
            pl.BlockSpec(memory_space=pltpu.VMEM),
        ],
        out_specs=(
            pl.BlockSpec(memory_space=pltpu.VMEM),
            pl.BlockSpec(memory_space=pltpu.VMEM),
        ),
    )(deg, v)


def _agg_body(src_hbm, dst_hbm, p_hbm, a_hbm, sidx_v, didx_v, vals_v, node_a, p_sh, a_sh):
    c = lax.axis_index("c")
    s = lax.axis_index("s")
    w = c * NS + s
    nbase = pl.multiple_of(s * SL, SL)

    def zero_i(i, _):
        node_a[pl.ds(i * 16, 16)] = jnp.zeros((16,), _f32)
        return 0
    lax.fori_loop(0, SL // 16, zero_i, 0)

    pltpu.sync_copy(node_a, a_sh.at[pl.ds(nbase, SL)])
    pltpu.sync_copy(p_hbm.at[pl.ds(nbase, SL)], node_a)
    pltpu.sync_copy(node_a, p_sh.at[pl.ds(nbase, SL)])
    plsc.subcore_barrier()

    def agg_step(k, _):
        off = pl.multiple_of(w * EP + k * EC, EC)
        pltpu.sync_copy(src_hbm.at[pl.ds(off, EC)], sidx_v)
        pltpu.sync_copy(dst_hbm.at[pl.ds(off, EC)], didx_v)
        pltpu.sync_copy(p_sh.at[sidx_v], vals_v)
        pltpu.sync_copy(vals_v, a_sh.at[didx_v], add=True)
        return 0
    lax.fori_loop(0, EP // EC, agg_step, 0)
    plsc.subcore_barrier()

    pltpu.sync_copy(a_sh.at[pl.ds(nbase, SL)], node_a)
    pltpu.sync_copy(node_a, a_hbm.at[c, pl.ds(nbase, SL)])


def _agg_kernel(src, dst, p):
    f = pl.kernel(
        _agg_body,
        out_type=jax.ShapeDtypeStruct((NC, NP), _f32),
        mesh=_mesh(),
        scratch_types=[
            pltpu.VMEM((EC,), jnp.int32),
            pltpu.VMEM((EC,), jnp.int32),
            pltpu.VMEM((EC,), _f32),
            pltpu.VMEM((SL,), _f32),
            pltpu.VMEM_SHARED((NP,), _f32),
            pltpu.VMEM_SHARED((NP,), _f32),
        ],
    )
    return f(src, dst, p)


def _fin_body(a_ref, p_ref, dinv_ref, cst_ref, out_ref):
    out_ref[...] = dinv_ref[...] * (a_ref[0] + a_ref[1] + p_ref[...]) + cst_ref[0]


def _fin(a_part, p, dinv, cst):
    return pl.pallas_call(
        _fin_body,
        out_shape=jax.ShapeDtypeStruct((ROWS, 128), _f32),
        in_specs=[
            pl.BlockSpec(memory_space=pltpu.VMEM),
            pl.BlockSpec(memory_space=pltpu.VMEM),
            pl.BlockSpec(memory_space=pltpu.VMEM),
            pl.BlockSpec(memory_space=pltpu.SMEM),
        ],
        out_specs=pl.BlockSpec(memory_space=pltpu.VMEM),
    )(a_part, p, dinv, cst)


@jax.jit
def kernel(x, edge_index, W1, b1, gamma, beta, alpha, W2, b2, Wg, bg, Ww, bw):
    wgw = Wg @ Ww
    wv = (W2 @ wgw)[:, 0]
    cv = (b2 @ wgw)[0]
    const = (bg @ Ww)[0] + bw[0]
    a0 = gamma * W1[0]
    a1 = gamma * W1[1]
    cc = gamma * b1 + beta
    coef = jnp.concatenate([
        a0, a1, cc, wv,
        jnp.reshape(alpha, (1,)), jnp.reshape(cv, (1,)),
    ]).astype(_f32)

    x0 = jnp.pad(x[:, 0], (0, NP - N)).reshape(ROWS, 128)
    x1 = jnp.pad(x[:, 1], (0, NP - N)).reshape(ROWS, 128)
    v = _mlp(x0, x1, coef)

    deg_part, src_flat, dst_flat = _deg_kernel(edge_index)
    p, dinv = _dp(deg_part.reshape(NC, ROWS, 128), v)

    a_part = _agg_kernel(src_flat, dst_flat, p.reshape(NP))
    scores = _fin(a_part.reshape(NC, ROWS, 128), p, dinv,
                  jnp.reshape(const, (1,)))
    return scores.reshape(NP)[:N]


# speedup vs baseline: 310.8026x; 1.0258x over previous
import jax
import jax.numpy as jnp
from jax import lax
from jax.experimental import pallas as pl
from jax.experimental.pallas import tpu as pltpu
from jax.experimental.pallas import tpu_sc as plsc

N = 100000
E = 1600000
H = 32
NC = 2
NS = 16
NW = NC * NS
NP = 100352
SL = NP // NS
ECE = 12800
NCH = E // ECE
EP = E // NW
EC = 10000
ROWS = NP // 128

_f32 = jnp.float32


def _mesh():
    return plsc.VectorSubcoreMesh(
        core_axis_name="c", subcore_axis_name="s", num_cores=NC)


def _nchunks(w):
    return 3 + jnp.where(w < NCH - 3 * NW, 1, 0)


def _deg_body(ei_hbm, deg_hbm, src_hbm, dst_hbm,
              eiv0, eiv1, sidx_v, didx_v, vals_v, node_a, deg_sh,
              seml0, seml1, semw):
    c = lax.axis_index("c")
    s = lax.axis_index("s")
    w = c * NS + s
    nbase = pl.multiple_of(s * SL, SL)

    def init_i(i, _):
        vals_v[pl.ds(i * 16, 16)] = jnp.full((16,), 1.0, _f32)
        return 0
    lax.fori_loop(0, ECE // 16, init_i, 0)

    def zero_i(i, _):
        node_a[pl.ds(i * 16, 16)] = jnp.zeros((16,), _f32)
        return 0
    lax.fori_loop(0, SL // 16, zero_i, 0)

    pltpu.sync_copy(node_a, deg_sh.at[pl.ds(nbase, SL)])
    plsc.subcore_barrier()

    nc = _nchunks(w)

    def _off(k):
        return pl.multiple_of((w + NW * k) * ECE, 512)

    def _deint(buf):
        def dei(i, _):
            sl = pl.ds(pl.multiple_of(i * 16, 16), 16)
            sidx_v[sl] = buf[0, sl]
            didx_v[sl] = buf[1, sl]
            return 0
        lax.fori_loop(0, ECE // 16, dei, 0)

    def _tail(k):
        wa = pltpu.async_copy(sidx_v, src_hbm.at[pl.ds(_off(k), ECE)], semw)
        wb = pltpu.async_copy(didx_v, dst_hbm.at[pl.ds(_off(k), ECE)], semw)
        pltpu.sync_copy(vals_v, deg_sh.at[didx_v], add=True)
        wa.wait()
        wb.wait()

    bufs = (eiv0, eiv1)
    sems = (seml0, seml1)
    lds = [pltpu.async_copy(ei_hbm.at[:, pl.ds(_off(k), ECE)], bufs[k], sems[k])
           for k in range(2)]
    lds[0].wait()
    _deint(eiv0)
    ld2 = pltpu.async_copy(ei_hbm.at[:, pl.ds(_off(2), ECE)], eiv0, seml0)
    _tail(0)
    lds[1].wait()
    _deint(eiv1)

    @pl.when(nc == 4)
    def _():
        pltpu.async_copy(ei_hbm.at[:, pl.ds(_off(3), ECE)], eiv1, seml1)
    _tail(1)
    ld2.wait()
    _deint(eiv0)
    _tail(2)

    @pl.when(nc == 4)
    def _():
        pltpu.make_async_copy(
            ei_hbm.at[:, pl.ds(_off(3), ECE)], eiv1, seml1).wait()
        _deint(eiv1)
        _tail(3)
    plsc.subcore_barrier()

    pltpu.sync_copy(deg_sh.at[pl.ds(nbase, SL)], node_a)
    pltpu.sync_copy(node_a, deg_hbm.at[c, pl.ds(nbase, SL)])


def _deg_kernel(edge_index):
    f = pl.kernel(
        _deg_body,
        out_type=(
            jax.ShapeDtypeStruct((NC, NP), _f32),
            jax.ShapeDtypeStruct((E,), jnp.int32),
            jax.ShapeDtypeStruct((E,), jnp.int32),
        ),
        mesh=_mesh(),
        scratch_types=[
            pltpu.VMEM((2, ECE), jnp.int32),
            pltpu.VMEM((2, ECE), jnp.int32),
            pltpu.VMEM((ECE,), jnp.int32),
            pltpu.VMEM((ECE,), jnp.int32),
            pltpu.VMEM((ECE,), _f32),
            pltpu.VMEM((SL,), _f32),
            pltpu.VMEM_SHARED((NP,), _f32),
            pltpu.SemaphoreType.DMA,
            pltpu.SemaphoreType.DMA,
            pltpu.SemaphoreType.DMA,
        ],
    )
    return f(edge_index)


def _mlp_body(x0_ref, x1_ref, coef_ref, v_ref):
    x0 = x0_ref[...]
    x1 = x1_ref[...]
    alpha = coef_ref[4 * H]
    cv = coef_ref[4 * H + 1]
    acc = jnp.full_like(x0, cv)
    for j in range(H):
        t = x0 * coef_ref[j] + x1 * coef_ref[H + j] + coef_ref[2 * H + j]
        t = jnp.where(t > 0, t, alpha * t)
        acc = acc + t * coef_ref[3 * H + j]
    v_ref[...] = acc


def _mlp(x0, x1, coef):
    return pl.pallas_call(
        _mlp_body,
        out_shape=jax.ShapeDtypeStruct((ROWS, 128), _f32),
        in_specs=[
            pl.BlockSpec(memory_space=pltpu.VMEM),
            pl.BlockSpec(memory_space=pltpu.VMEM),
            pl.BlockSpec(memory_space=pltpu.SMEM),
        ],
        out_specs=pl.BlockSpec(memory_space=pltpu.VMEM),
    )(x0, x1, coef)


def _dp_body(deg_ref, v_ref, p_ref, dinv_ref):
    d = deg_ref[0] + deg_ref[1] + 1.0
    dinv = lax.rsqrt(d)
    dinv = dinv * (1.5 - 0.5 * d * dinv * dinv)
    dinv_ref[...] = dinv
    p_ref[...] = v_ref[...] * dinv


def _dp(deg, v):
    return pl.pallas_call(
        _dp_body,
        out_shape=(
            jax.ShapeDtypeStruct((ROWS, 128), _f32),
            jax.ShapeDtypeStruct((ROWS, 128), _f32),
        ),
        in_specs=[
            pl.BlockSpec(memory_space=pltpu.VMEM),
            pl.BlockSpec(memory_space=pltpu.VMEM),
        ],
        out_specs=(
            pl.BlockSpec(memory_space=pltpu.VMEM),
            pl.BlockSpec(memory_space=pltpu.VMEM),
        ),
    )(deg, v)


def _agg_body(src_hbm, dst_hbm, p_hbm, a_hbm,
              sidx0, sidx1, didx0, didx1, vals_v, node_a, p_sh, a_sh,
              seml0, seml1):
    c = lax.axis_index("c")
    s = lax.axis_index("s")
    w = c * NS + s
    nbase = pl.multiple_of(s * SL, SL)

    def zero_i(i, _):
        node_a[pl.ds(i * 16, 16)] = jnp.zeros((16,), _f32)
        return 0
    lax.fori_loop(0, SL // 16, zero_i, 0)

    pltpu.sync_copy(node_a, a_sh.at[pl.ds(nbase, SL)])
    pltpu.sync_copy(p_hbm.at[pl.ds(nbase, SL)], node_a)
    pltpu.sync_copy(node_a, p_sh.at[pl.ds(nbase, SL)])
    plsc.subcore_barrier()

    def _off(k):
        return pl.multiple_of(w * EP + k * EC, EC)

    sbufs = (sidx0, sidx1)
    dbufs = (didx0, didx1)
    sems = (seml0, seml1)

    def _load(k):
        b = k % 2
        return (pltpu.async_copy(src_hbm.at[pl.ds(_off(k), EC)], sbufs[b], sems[b]),
                pltpu.async_copy(dst_hbm.at[pl.ds(_off(k), EC)], dbufs[b], sems[b]))

    nk = EP // EC
    pend = _load(0)
    for k in range(nk):
        b = k % 2
        for d in pend:
            d.wait()
        if k + 1 < nk:
            pend = _load(k + 1)
        pltpu.sync_copy(p_sh.at[sbufs[b]], vals_v)
        pltpu.sync_copy(vals_v, a_sh.at[dbufs[b]], add=True)
    plsc.subcore_barrier()

    pltpu.sync_copy(a_sh.at[pl.ds(nbase, SL)], node_a)
    pltpu.sync_copy(node_a, a_hbm.at[c, pl.ds(nbase, SL)])


def _agg_kernel(src, dst, p):
    f = pl.kernel(
        _agg_body,
        out_type=jax.ShapeDtypeStruct((NC, NP), _f32),
        mesh=_mesh(),
        scratch_types=[
            pltpu.VMEM((EC,), jnp.int32),
            pltpu.VMEM((EC,), jnp.int32),
            pltpu.VMEM((EC,), jnp.int32),
            pltpu.VMEM((EC,), jnp.int32),
            pltpu.VMEM((EC,), _f32),
            pltpu.VMEM((SL,), _f32),
            pltpu.VMEM_SHARED((NP,), _f32),
            pltpu.VMEM_SHARED((NP,), _f32),
            pltpu.SemaphoreType.DMA,
            pltpu.SemaphoreType.DMA,
        ],
    )
    return f(src, dst, p)


def _fin_body(a_ref, p_ref, dinv_ref, cst_ref, out_ref):
    out_ref[...] = dinv_ref[...] * (a_ref[0] + a_ref[1] + p_ref[...]) + cst_ref[0]


def _fin(a_part, p, dinv, cst):
    return pl.pallas_call(
        _fin_body,
        out_shape=jax.ShapeDtypeStruct((ROWS, 128), _f32),
        in_specs=[
            pl.BlockSpec(memory_space=pltpu.VMEM),
            pl.BlockSpec(memory_space=pltpu.VMEM),
            pl.BlockSpec(memory_space=pltpu.VMEM),
            pl.BlockSpec(memory_space=pltpu.SMEM),
        ],
        out_specs=pl.BlockSpec(memory_space=pltpu.VMEM),
    )(a_part, p, dinv, cst)


@jax.jit
def kernel(x, edge_index, W1, b1, gamma, beta, alpha, W2, b2, Wg, bg, Ww, bw):
    wgw = Wg @ Ww
    wv = (W2 @ wgw)[:, 0]
    cv = (b2 @ wgw)[0]
    const = (bg @ Ww)[0] + bw[0]
    a0 = gamma * W1[0]
    a1 = gamma * W1[1]
    cc = gamma * b1 + beta
    coef = jnp.concatenate([
        a0, a1, cc, wv,
        jnp.reshape(alpha, (1,)), jnp.reshape(cv, (1,)),
    ]).astype(_f32)

    x0 = jnp.pad(x[:, 0], (0, NP - N)).reshape(ROWS, 128)
    x1 = jnp.pad(x[:, 1], (0, NP - N)).reshape(ROWS, 128)
    v = _mlp(x0, x1, coef)

    deg_part, src_flat, dst_flat = _deg_kernel(edge_index)
    p, dinv = _dp(deg_part.reshape(NC, ROWS, 128), v)

    a_part = _agg_kernel(src_flat, dst_flat, p.reshape(NP))
    scores = _fin(a_part.reshape(NC, ROWS, 128), p, dinv,
                  jnp.reshape(const, (1,)))
    return scores.reshape(NP)[:N]
